# baseline (speedup 1.0000x reference)
import jax
import jax.numpy as jnp
from jax import lax
from jax.experimental import pallas as pl
from jax.experimental.pallas import tpu as pltpu
from jax.experimental.pallas import tpu_sc as plsc

_B = 1024
_N = 100000
_MARGIN = 1.0
_MN = -50.0
_MX = 50.0
_TAU = 0.1

_NC = 2
_NSUB = 16
_NW = _NC * _NSUB
_RW = _B // _NW
_C = 10000
_NCH = _N // _C
_L = 16
_U = 5
_STEPS = _C // (_L * _U)


def _insert(a1, a2, a3, x):
    b1 = jnp.maximum(a1, x)
    r1 = jnp.minimum(a1, x)
    b2 = jnp.maximum(a2, r1)
    r2 = jnp.minimum(a2, r1)
    b3 = jnp.maximum(a3, r2)
    return b1, b2, b3


def _splat(x):
    return jnp.full((_L,), x, jnp.float32)


def _sc_body(sim_hbm, tgt_hbm, out_hbm,
             sim0, sim1, tgt0, tgt1, res_v, sem0, sem1):
    cid = lax.axis_index("c")
    sid = lax.axis_index("s")
    wid = cid * _NSUB + sid
    row0 = wid * _RW

    sims = (sim0, sim1)
    tgts = (tgt0, tgt1)
    sems = (sem0, sem1)

    def _off(r, ch):
        return pl.multiple_of(r * _N + ch * _C, 8)

    def start(r, ch, par):
        pltpu.async_copy(sim_hbm.at[pl.ds(_off(r, ch), _C)], sims[par], sems[par])
        pltpu.async_copy(tgt_hbm.at[pl.ds(_off(r, ch), _C)], tgts[par], sems[par])

    def wait(r, ch, par):
        pltpu.make_async_copy(sim_hbm.at[pl.ds(_off(r, ch), _C)], sims[par], sems[par]).wait()
        pltpu.make_async_copy(tgt_hbm.at[pl.ds(_off(r, ch), _C)], tgts[par], sems[par]).wait()

    start(row0, 0, 0)

    def row_body(rl, carry):
        r = row0 + rl
        a1 = [_splat(_MN)] * _U
        a2 = [_splat(_MN)] * _U
        a3 = [_splat(_MN)] * _U
        pa = [_splat(_MX)] * _U
        acc = tuple(a1 + a2 + a3 + pa)

        for ch in range(_NCH):
            par = ch % 2
            wait(r, ch, par)
            if ch < _NCH - 1:
                start(r, ch + 1, 1 - par)
            else:
                @pl.when(rl + 1 < _RW)
                def _():
                    start(r + 1, 0, 0)

            sbuf = sims[par]
            tbuf = tgts[par]

            def step(j, acc):
                acc = list(acc)
                base = j * (_L * _U)
                for u in range(_U):
                    x = sbuf[pl.ds(base + u * _L, _L)]
                    t = tbuf[pl.ds(base + u * _L, _L)]
                    eq = t == 1
                    xm = jnp.where(eq, jnp.float32(_MN), x)
                    xp = jnp.where(eq, x, jnp.float32(_MX))
                    acc[3 * _U + u] = jnp.minimum(acc[3 * _U + u], xp)
                    acc[u], acc[_U + u], acc[2 * _U + u] = _insert(
                        acc[u], acc[_U + u], acc[2 * _U + u], xm)
                return tuple(acc)

            acc = lax.fori_loop(0, _STEPS, step, acc)

        acc = list(acc)
        A1, A2, A3 = acc[0], acc[_U], acc[2 * _U]
        P = acc[3 * _U]
        for u in range(1, _U):
            A1, A2, A3 = _insert(A1, A2, A3, acc[u])
            A1, A2, A3 = _insert(A1, A2, A3, acc[_U + u])
            A1, A2, A3 = _insert(A1, A2, A3, acc[2 * _U + u])
            P = jnp.minimum(P, acc[3 * _U + u])

        res_v[pl.ds(0, _L)] = A1
        res_v[pl.ds(_L, _L)] = A2
        res_v[pl.ds(2 * _L, _L)] = A3
        res_v[pl.ds(3 * _L, _L)] = P
        pltpu.sync_copy(
            res_v, out_hbm.at[pl.ds(pl.multiple_of(r * 4 * _L, 8), 4 * _L)])
        return carry

    lax.fori_loop(0, _RW, row_body, jnp.int32(0))


def _tc_finalize(x_ref, o_ref):
    x = x_ref[...]
    col = lax.broadcasted_iota(jnp.int32, x.shape, 1)
    neg = jnp.float32(-1e30)
    top = jnp.where(col < 3 * _L, x, neg)
    pos = jnp.where(col >= 3 * _L, x, jnp.float32(1e30))
    p = jnp.min(pos, axis=1, keepdims=True)
    m1 = jnp.max(top, axis=1, keepdims=True)
    c1 = jnp.sum(jnp.where(top == m1, 1.0, 0.0), axis=1, keepdims=True)
    w2 = jnp.where(top < m1, top, neg)
    m2 = jnp.max(w2, axis=1, keepdims=True)
    c2 = jnp.sum(jnp.where(top == m2, 1.0, 0.0), axis=1, keepdims=True)
    w3 = jnp.where(top < m2, top, neg)
    m3 = jnp.max(w3, axis=1, keepdims=True)
    v1 = m1
    v2 = jnp.where(c1 >= 2.0, m1, m2)
    v3 = jnp.where(c1 >= 3.0, m1,
                   jnp.where(jnp.logical_or(c1 == 2.0, c2 >= 2.0), m2, m3))
    itau = jnp.float32(1.0 / _TAU)
    e1 = jnp.exp((v1 - m1) * itau)
    e2 = jnp.exp((v2 - m1) * itau)
    e3 = jnp.exp((v3 - m1) * itau)
    mg = jnp.float32(_MARGIN)
    l1 = jnp.maximum(v1 - p + mg, 0.0)
    l2 = jnp.maximum(v2 - p + mg, 0.0)
    l3 = jnp.maximum(v3 - p + mg, 0.0)
    row_loss = (l1 * e1 + l2 * e2 + l3 * e3) / (e1 + e2 + e3)
    o_ref[...] = (jnp.sum(row_loss) * jnp.float32(1.0 / (_B * 3.0)))[None, None]


@jax.jit
def kernel(sim_b, target):
    mesh = plsc.VectorSubcoreMesh(
        core_axis_name="c", subcore_axis_name="s",
        num_cores=_NC, num_subcores=_NSUB)
    survivors = pl.kernel(
        _sc_body,
        out_type=jax.ShapeDtypeStruct((_B * 4 * _L,), jnp.float32),
        mesh=mesh,
        scratch_types=[
            pltpu.VMEM((_C,), jnp.float32),
            pltpu.VMEM((_C,), jnp.float32),
            pltpu.VMEM((_C,), jnp.int32),
            pltpu.VMEM((_C,), jnp.int32),
            pltpu.VMEM((4 * _L,), jnp.float32),
            pltpu.SemaphoreType.DMA,
            pltpu.SemaphoreType.DMA,
        ],
    )(sim_b.reshape(-1), target.reshape(-1))
    total = pl.pallas_call(
        _tc_finalize,
        out_shape=jax.ShapeDtypeStruct((1, 1), jnp.float32),
    )(survivors.reshape(_B, 4 * _L))
    return total[0, 0]


# speedup vs baseline: 1.8016x; 1.8016x over previous
import jax
import jax.numpy as jnp
from jax import lax
from jax.experimental import pallas as pl
from jax.experimental.pallas import tpu as pltpu
from jax.experimental.pallas import tpu_sc as plsc

_B = 1024
_N = 100000
_MARGIN = 1.0
_MN = -50.0
_MX = 50.0
_TAU = 0.1

_NC = 2
_NSUB = 16
_NW = _NC * _NSUB
_L = 16
_T = 128
_NT = _N // _T
_TAIL = _NT * _T
_GK = (_B // 8) // _NW
_K = 11
_KW = 22
_NCH = 70
_PAIRS = 34


def _insert(a1, a2, a3, x):
    b1 = jnp.maximum(a1, x)
    r1 = jnp.minimum(a1, x)
    b2 = jnp.maximum(a2, r1)
    r2 = jnp.minimum(a2, r1)
    b3 = jnp.maximum(a3, r2)
    return b1, b2, b3


def _splat(x):
    return jnp.full((_L,), x, jnp.float32)


def _sc_body(sim_hbm, tgt_hbm, out_hbm,
             s0, s1, t0, t1, sw, tw, res_v, sem0, sem1, semw):
    cid = lax.axis_index("c")
    sid = lax.axis_index("s")
    wid = cid * _NSUB + sid
    g0 = wid * _GK

    sbufs = (s0, s1)
    tbufs = (t0, t1)
    sems = (sem0, sem1)

    def _slc(g, tile0, ntiles):
        row = pl.multiple_of((g0 + g) * 8, 8)
        col = pl.multiple_of(tile0 * _T, _T)
        return (pl.ds(row, 8), pl.ds(col, ntiles * _T))

    def start11(g, c, par):
        idx = _slc(g, c * _K, _K)
        pltpu.async_copy(sim_hbm.at[idx], sbufs[par], sems[par])
        pltpu.async_copy(tgt_hbm.at[idx], tbufs[par], sems[par])

    def wait11(g, c, par):
        idx = _slc(g, c * _K, _K)
        pltpu.make_async_copy(sim_hbm.at[idx], sbufs[par], sems[par]).wait()
        pltpu.make_async_copy(tgt_hbm.at[idx], tbufs[par], sems[par]).wait()

    def startw(g):
        idx = _slc(g, 69 * _K, _KW)
        pltpu.async_copy(sim_hbm.at[idx], sw, semw)
        pltpu.async_copy(tgt_hbm.at[idx], tw, semw)

    def waitw(g):
        idx = _slc(g, 69 * _K, _KW)
        pltpu.make_async_copy(sim_hbm.at[idx], sw, semw).wait()
        pltpu.make_async_copy(tgt_hbm.at[idx], tw, semw).wait()

    def proc(sbuf, tbuf, ntiles, acc):
        acc = list(acc)
        for r8 in range(8):
            def tloop(t, sub, r8=r8):
                a1, a2, a3, pa = sub
                for h in range(8):
                    off = t * _T + h * _L
                    x = sbuf[r8, pl.ds(off, _L)]
                    tt = tbuf[r8, pl.ds(off, _L)]
                    eq = tt == 1
                    xm = jnp.where(eq, jnp.float32(_MN), x)
                    xp = jnp.where(eq, x, jnp.float32(_MX))
                    pa = jnp.minimum(pa, xp)
                    a1, a2, a3 = _insert(a1, a2, a3, xm)
                return (a1, a2, a3, pa)
            acc[r8 * 4:(r8 + 1) * 4] = lax.fori_loop(
                0, ntiles, tloop, tuple(acc[r8 * 4:(r8 + 1) * 4]))
        return tuple(acc)

    start11(0, 0, 0)

    def group_body(gi, carry):
        acc = tuple([_splat(_MN), _splat(_MN), _splat(_MN), _splat(_MX)] * 8)

        def pair_body(j, acc):
            c = 2 * j
            wait11(gi, c, 0)
            start11(gi, c + 1, 1)
            acc = proc(s0, t0, _K, acc)
            wait11(gi, c + 1, 1)
            start11(gi, c + 2, 0)
            acc = proc(s1, t1, _K, acc)
            return acc

        acc = lax.fori_loop(0, _PAIRS, pair_body, acc)
        startw(gi)
        wait11(gi, 68, 0)
        acc = proc(s0, t0, _K, acc)

        @pl.when(gi < _GK - 1)
        def _():
            start11(gi + 1, 0, 0)

        waitw(gi)
        acc = proc(sw, tw, _KW, acc)

        for r8 in range(8):
            a1, a2, a3, pa = acc[r8 * 4:(r8 + 1) * 4]
            res_v[pl.ds(r8 * 4 * _L, _L)] = a1
            res_v[pl.ds(r8 * 4 * _L + _L, _L)] = a2
            res_v[pl.ds(r8 * 4 * _L + 2 * _L, _L)] = a3
            res_v[pl.ds(r8 * 4 * _L + 3 * _L, _L)] = pa
        off = pl.multiple_of((g0 + gi) * 8 * 4 * _L, 8)
        pltpu.sync_copy(res_v, out_hbm.at[pl.ds(off, 8 * 4 * _L)])
        return carry

    lax.fori_loop(0, _GK, group_body, jnp.int32(0))


def _tc_finalize(s_ref, st_ref, tt_ref, o_ref):
    s = s_ref[...]
    stail = st_ref[...]
    ttail = tt_ref[...]
    eqt = ttail == 1
    tail_top = jnp.where(eqt, jnp.float32(_MN), stail)
    tail_pos = jnp.where(eqt, stail, jnp.float32(_MX))
    top = jnp.concatenate([s[:, :3 * _L], tail_top], axis=1)
    posm = jnp.concatenate([s[:, 3 * _L:], tail_pos], axis=1)

    neg = jnp.float32(-1e30)
    p = jnp.min(posm, axis=1, keepdims=True)
    m1 = jnp.max(top, axis=1, keepdims=True)
    c1 = jnp.sum(jnp.where(top == m1, 1.0, 0.0), axis=1, keepdims=True)
    w2 = jnp.where(top < m1, top, neg)
    m2 = jnp.max(w2, axis=1, keepdims=True)
    c2 = jnp.sum(jnp.where(top == m2, 1.0, 0.0), axis=1, keepdims=True)
    w3 = jnp.where(top < m2, top, neg)
    m3 = jnp.max(w3, axis=1, keepdims=True)
    v1 = m1
    v2 = jnp.where(c1 >= 2.0, m1, m2)
    v3 = jnp.where(c1 >= 3.0, m1,
                   jnp.where(jnp.logical_or(c1 == 2.0, c2 >= 2.0), m2, m3))
    itau = jnp.float32(1.0 / _TAU)
    e1 = jnp.exp((v1 - m1) * itau)
    e2 = jnp.exp((v2 - m1) * itau)
    e3 = jnp.exp((v3 - m1) * itau)
    mg = jnp.float32(_MARGIN)
    l1 = jnp.maximum(v1 - p + mg, 0.0)
    l2 = jnp.maximum(v2 - p + mg, 0.0)
    l3 = jnp.maximum(v3 - p + mg, 0.0)
    row_loss = (l1 * e1 + l2 * e2 + l3 * e3) / (e1 + e2 + e3)
    o_ref[...] = (jnp.sum(row_loss) * jnp.float32(1.0 / (_B * 3.0)))[None, None]


@jax.jit
def kernel(sim_b, target):
    mesh = plsc.VectorSubcoreMesh(
        core_axis_name="c", subcore_axis_name="s",
        num_cores=_NC, num_subcores=_NSUB)
    survivors = pl.kernel(
        _sc_body,
        out_type=jax.ShapeDtypeStruct((_B * 4 * _L,), jnp.float32),
        mesh=mesh,
        scratch_types=[
            pltpu.VMEM((8, _K * _T), jnp.float32),
            pltpu.VMEM((8, _K * _T), jnp.float32),
            pltpu.VMEM((8, _K * _T), jnp.int32),
            pltpu.VMEM((8, _K * _T), jnp.int32),
            pltpu.VMEM((8, _KW * _T), jnp.float32),
            pltpu.VMEM((8, _KW * _T), jnp.int32),
            pltpu.VMEM((8 * 4 * _L,), jnp.float32),
            pltpu.SemaphoreType.DMA,
            pltpu.SemaphoreType.DMA,
            pltpu.SemaphoreType.DMA,
        ],
    )(sim_b, target)
    total = pl.pallas_call(
        _tc_finalize,
        out_shape=jax.ShapeDtypeStruct((1, 1), jnp.float32),
    )(survivors.reshape(_B, 4 * _L), sim_b[:, _TAIL:], target[:, _TAIL:])
    return total[0, 0]
